# Initial kernel scaffold; baseline (speedup 1.0000x reference)
import jax
import jax.numpy as jnp
from jax.experimental import pallas as pl


def kernel(x, amino_table, element_table, position_table, W_xyz, b_xyz):
    raise NotImplementedError("write your pallas kernel here")


# speedup vs baseline: 2.2861x; 2.2861x over previous
import functools

import jax
import jax.numpy as jnp
from jax import lax
from jax.experimental import pallas as pl
from jax.experimental.pallas import tpu as pltpu

_SF = 2
_D = 28 * _SF
_DX = 172 * _SF
_DOUT = 3 * _D + _DX
_K = 64
_BN = 1024


def _fused_body(x_ref, w_ref, out_ref):
    xb = x_ref[...]
    res = xb[:, 0:1].astype(jnp.int32)
    atom = xb[:, 1:2].astype(jnp.int32)
    cnt = xb[:, 2:3].astype(jnp.int32)

    c = lax.broadcasted_iota(jnp.int32, (_BN, _K), 1)
    onehot = ((c == res) & (c < 20)) | ((c - 20 == atom) & (c >= 20) & (c < 25))
    onehot |= (c - 25 == cnt) & (c >= 25) & (c < 46)
    feats = onehot.astype(jnp.float32)
    feats += jnp.where(c == 46, xb[:, 3:4], 0.0)
    feats += jnp.where(c == 47, xb[:, 4:5], 0.0)
    feats += jnp.where(c == 48, xb[:, 5:6], 0.0)
    feats += (c == 49).astype(jnp.float32)

    y = jnp.dot(feats, w_ref[...], preferred_element_type=jnp.float32,
                precision=lax.Precision.HIGHEST)
    c_out = lax.broadcasted_iota(jnp.int32, (_BN, _DOUT), 1)
    out_ref[...] = jnp.where(c_out >= 3 * _D, jnp.maximum(y, 0.0), y)


@functools.partial(jax.jit, static_argnames=())
def _pack_weights(amino_table, element_table, position_table, W_xyz, b_xyz):
    w = jnp.zeros((_K, _DOUT), dtype=jnp.float32)
    w = w.at[0:20, 0:_D].set(amino_table)
    w = w.at[20:25, _D:2 * _D].set(element_table)
    w = w.at[25:46, 2 * _D:3 * _D].set(position_table)
    w = w.at[46:49, 3 * _D:].set(W_xyz)
    w = w.at[49, 3 * _D:].set(b_xyz)
    return w


def kernel(x, amino_table, element_table, position_table, W_xyz, b_xyz):
    B, L, _ = x.shape
    n = B * L
    xf = x.reshape(n, 6)
    w = _pack_weights(amino_table, element_table, position_table, W_xyz, b_xyz)
    out = pl.pallas_call(
        _fused_body,
        grid=(n // _BN,),
        in_specs=[
            pl.BlockSpec((_BN, 6), lambda i: (i, 0)),
            pl.BlockSpec((_K, _DOUT), lambda i: (0, 0)),
        ],
        out_specs=pl.BlockSpec((_BN, _DOUT), lambda i: (i, 0)),
        out_shape=jax.ShapeDtypeStruct((n, _DOUT), jnp.float32),
        compiler_params=pltpu.CompilerParams(
            dimension_semantics=("parallel",),
        ),
    )(xf, w)
    return out.reshape(B, L, _DOUT)


# speedup vs baseline: 2.9028x; 1.2698x over previous
import functools

import jax
import jax.numpy as jnp
from jax import lax
from jax.experimental import pallas as pl
from jax.experimental.pallas import tpu as pltpu

_SF = 2
_D = 28 * _SF
_DX = 172 * _SF
_DOUT = 3 * _D + _DX
_K = 64
_BN = 1024


def _fused_body(x_ref, w_ref, out_ref):
    xb = x_ref[...]
    res = xb[:, 0:1].astype(jnp.int32)
    atom = xb[:, 1:2].astype(jnp.int32)
    cnt = xb[:, 2:3].astype(jnp.int32)

    c = lax.broadcasted_iota(jnp.int32, (_BN, _K), 1)
    onehot = ((c == res) & (c < 20)) | ((c - 20 == atom) & (c >= 20) & (c < 25))
    onehot |= (c - 25 == cnt) & (c >= 25) & (c < 46)
    feats = onehot.astype(jnp.float32)
    feats += jnp.where(c == 46, xb[:, 3:4], 0.0)
    feats += jnp.where(c == 47, xb[:, 4:5], 0.0)
    feats += jnp.where(c == 48, xb[:, 5:6], 0.0)
    feats += (c == 49).astype(jnp.float32)

    y = jnp.dot(feats, w_ref[...], preferred_element_type=jnp.float32,
                precision=lax.Precision.DEFAULT)
    c_out = lax.broadcasted_iota(jnp.int32, (_BN, _DOUT), 1)
    out_ref[...] = jnp.where(c_out >= 3 * _D, jnp.maximum(y, 0.0), y)


@functools.partial(jax.jit, static_argnames=())
def _pack_weights(amino_table, element_table, position_table, W_xyz, b_xyz):
    w = jnp.zeros((_K, _DOUT), dtype=jnp.float32)
    w = w.at[0:20, 0:_D].set(amino_table)
    w = w.at[20:25, _D:2 * _D].set(element_table)
    w = w.at[25:46, 2 * _D:3 * _D].set(position_table)
    w = w.at[46:49, 3 * _D:].set(W_xyz)
    w = w.at[49, 3 * _D:].set(b_xyz)
    return w


def kernel(x, amino_table, element_table, position_table, W_xyz, b_xyz):
    B, L, _ = x.shape
    n = B * L
    xf = x.reshape(n, 6)
    w = _pack_weights(amino_table, element_table, position_table, W_xyz, b_xyz)
    out = pl.pallas_call(
        _fused_body,
        grid=(n // _BN,),
        in_specs=[
            pl.BlockSpec((_BN, 6), lambda i: (i, 0)),
            pl.BlockSpec((_K, _DOUT), lambda i: (0, 0)),
        ],
        out_specs=pl.BlockSpec((_BN, _DOUT), lambda i: (i, 0)),
        out_shape=jax.ShapeDtypeStruct((n, _DOUT), jnp.float32),
        compiler_params=pltpu.CompilerParams(
            dimension_semantics=("parallel",),
        ),
    )(xf, w)
    return out.reshape(B, L, _DOUT)


# speedup vs baseline: 4.2529x; 1.4651x over previous
import jax
import jax.numpy as jnp
from jax import lax
from jax.experimental import pallas as pl
from jax.experimental.pallas import tpu as pltpu

_SF = 2
_D = 28 * _SF
_DX = 172 * _SF
_DOUT = 3 * _D + _DX
_K = 64
_BB = 16


def _fused_body(x_ref, w_ref, wx_ref, out_ref):
    xb = x_ref[...]
    bb, l, _ = xb.shape
    t1 = xb[:, :, 0:1].astype(jnp.int32)
    t2 = xb[:, :, 1:2].astype(jnp.int32) + 20
    t3 = xb[:, :, 2:3].astype(jnp.int32) + 25

    c = lax.broadcasted_iota(jnp.int32, (bb, l, _K), 2)
    ones = (c == t1) | (c == t2) | (c == t3) | (c == 49)
    feats = ones.astype(jnp.float32)

    y = lax.dot_general(feats, w_ref[...], (((2,), (0,)), ((), ())),
                        preferred_element_type=jnp.float32)
    y += lax.dot_general(xb, wx_ref[...], (((2,), (0,)), ((), ())),
                         preferred_element_type=jnp.float32)
    out_ref[...] = jnp.maximum(y, w_ref[50:51, :][None])


def _pack_weights(amino_table, element_table, position_table, W_xyz, b_xyz):
    w = jnp.zeros((_K, _DOUT), dtype=jnp.float32)
    w = w.at[0:20, 0:_D].set(amino_table)
    w = w.at[20:25, _D:2 * _D].set(element_table)
    w = w.at[25:46, 2 * _D:3 * _D].set(position_table)
    w = w.at[49, 3 * _D:].set(b_xyz)
    w = w.at[50, 0:3 * _D].set(jnp.finfo(jnp.float32).min)
    wx = jnp.zeros((6, _DOUT), dtype=jnp.float32)
    wx = wx.at[3:6, 3 * _D:].set(W_xyz)
    return w, wx


def kernel(x, amino_table, element_table, position_table, W_xyz, b_xyz):
    B, L, _ = x.shape
    w, wx = _pack_weights(amino_table, element_table, position_table, W_xyz,
                          b_xyz)
    return pl.pallas_call(
        _fused_body,
        grid=(B // _BB,),
        in_specs=[
            pl.BlockSpec((_BB, L, 6), lambda i: (i, 0, 0)),
            pl.BlockSpec((_K, _DOUT), lambda i: (0, 0)),
            pl.BlockSpec((6, _DOUT), lambda i: (0, 0)),
        ],
        out_specs=pl.BlockSpec((_BB, L, _DOUT), lambda i: (i, 0, 0)),
        out_shape=jax.ShapeDtypeStruct((B, L, _DOUT), jnp.float32),
        compiler_params=pltpu.CompilerParams(
            dimension_semantics=("parallel",),
        ),
    )(x, w, wx)


# speedup vs baseline: 22.3556x; 5.2565x over previous
import jax
import jax.numpy as jnp
from jax import lax
from jax.experimental import pallas as pl
from jax.experimental.pallas import tpu as pltpu

_SF = 2
_D = 28 * _SF
_DX = 172 * _SF
_DOUT = 3 * _D + _DX
_K = 64
_BB = 128


def _fused_body(xt_ref, w_ref, out_ref):
    xt = xt_ref[...]
    _, l, bb = xt.shape
    t1 = xt[0:1].astype(jnp.int32)
    t2 = xt[1:2].astype(jnp.int32) + 20
    t3 = xt[2:3].astype(jnp.int32) + 25

    c = lax.broadcasted_iota(jnp.int32, (_K, l, bb), 0)
    ones = (c == t1) | (c == t2) | (c == t3) | (c == 49)
    feats = ones.astype(jnp.float32)
    feats += jnp.where(c == 46, xt[3:4], 0.0)
    feats += jnp.where(c == 47, xt[4:5], 0.0)
    feats += jnp.where(c == 48, xt[5:6], 0.0)

    y = lax.dot_general(feats, w_ref[...], (((0,), (0,)), ((), ())),
                        preferred_element_type=jnp.float32)
    out_ref[...] = jnp.maximum(y, w_ref[50:51, :][None])


def _pack_weights(amino_table, element_table, position_table, W_xyz, b_xyz):
    w = jnp.zeros((_K, _DOUT), dtype=jnp.float32)
    w = w.at[0:20, 0:_D].set(amino_table)
    w = w.at[20:25, _D:2 * _D].set(element_table)
    w = w.at[25:46, 2 * _D:3 * _D].set(position_table)
    w = w.at[46:49, 3 * _D:].set(W_xyz)
    w = w.at[49, 3 * _D:].set(b_xyz)
    w = w.at[50, 0:3 * _D].set(jnp.finfo(jnp.float32).min)
    return w


def kernel(x, amino_table, element_table, position_table, W_xyz, b_xyz):
    B, L, _ = x.shape
    w = _pack_weights(amino_table, element_table, position_table, W_xyz,
                      b_xyz)
    xt = jnp.transpose(x, (2, 1, 0))
    out_t = pl.pallas_call(
        _fused_body,
        grid=(B // _BB,),
        in_specs=[
            pl.BlockSpec((6, L, _BB), lambda i: (0, 0, i)),
            pl.BlockSpec((_K, _DOUT), lambda i: (0, 0)),
        ],
        out_specs=pl.BlockSpec((L, _BB, _DOUT), lambda i: (0, i, 0)),
        out_shape=jax.ShapeDtypeStruct((L, B, _DOUT), jnp.float32),
        compiler_params=pltpu.CompilerParams(
            dimension_semantics=("parallel",),
        ),
    )(xt, w)
    return jnp.transpose(out_t, (1, 0, 2))
